# baseline (speedup 1.0000x reference)
import jax
import jax.numpy as jnp
from jax.experimental import pallas as pl
from jax.experimental.pallas import tpu as pltpu

BT = 512
BI = 704


def _ffn_kernel(sched_ref, x_ref, w1_ref, w3_ref, w2_ref, o_ref):
    b = pl.program_id(0)
    io = pl.program_id(1)
    nb = pl.num_programs(0)
    active = sched_ref[nb + b]

    @pl.when(active == 1)
    def _():
        xb = x_ref[...]
        w1b = w1_ref[0].astype(jnp.bfloat16)
        w3b = w3_ref[0].astype(jnp.bfloat16)
        w2b = w2_ref[0].astype(jnp.bfloat16)
        dn = (((1,), (1,)), ((), ()))
        h1 = jax.lax.dot_general(xb, w1b, dn, preferred_element_type=jnp.float32)
        h3 = jax.lax.dot_general(xb, w3b, dn, preferred_element_type=jnp.float32)
        h = (h1 * jax.nn.sigmoid(h1) * h3).astype(jnp.bfloat16)
        contrib = jnp.dot(h, w2b, preferred_element_type=jnp.float32)

        @pl.when(io == 0)
        def _():
            o_ref[...] = contrib

        @pl.when(io > 0)
        def _():
            o_ref[...] += contrib


@jax.jit
def kernel(x, expert_indices, w1, w2, w3):
    T, D = x.shape
    A = expert_indices.shape[1]
    E, I, _ = w1.shape
    S = T * A
    NB = S // BT + E
    NI = I // BI

    e_flat = expert_indices.reshape(-1).astype(jnp.int32)
    onehot = (e_flat[:, None] == jnp.arange(E, dtype=jnp.int32)[None, :]
              ).astype(jnp.int32)
    cum = jnp.cumsum(onehot, axis=0)
    counts = cum[-1]
    blocks_per = (counts + BT - 1) // BT
    blocks_cum = jnp.cumsum(blocks_per)
    block_start = blocks_cum - blocks_per
    total_blocks = blocks_cum[-1]

    rank = jnp.take_along_axis(cum, e_flat[:, None], axis=1)[:, 0] - 1
    dest = block_start[e_flat] * BT + rank

    b_ids = jnp.arange(NB, dtype=jnp.int32)
    be_log = jnp.minimum(
        jnp.searchsorted(blocks_cum, b_ids, side="right").astype(jnp.int32), E - 1)

    j = jnp.arange(S, dtype=jnp.int32)
    tok_pad = jnp.zeros((NB * BT,), jnp.int32).at[dest].set(j // A)
    x_pad = x.astype(jnp.bfloat16)[tok_pad]

    half = NB // 2
    nb0 = jnp.minimum((total_blocks + 1) // 2, half)
    nb1 = total_blocks - nb0
    within = jnp.where(b_ids < half, b_ids, b_ids - half)
    base = jnp.where(b_ids < half, 0, nb0)
    half_n = jnp.where(b_ids < half, nb0, nb1)
    act = (within < half_n).astype(jnp.int32)
    last_log = jnp.minimum(base + jnp.maximum(half_n - 1, 0), total_blocks - 1)
    logical = jnp.where(act == 1, base + within, last_log)
    oblk = jnp.where(act == 1, logical, NB - 1)
    sched = jnp.concatenate([be_log[logical], act, logical, oblk])

    def w_map(b, io, s):
        return (s[b], jnp.where(s[NB + b] == 1, io, NI - 1), 0)

    def x_map(b, io, s):
        return (s[2 * NB + b], 0)

    def o_map(b, io, s):
        return (s[3 * NB + b], 0)

    grid_spec = pltpu.PrefetchScalarGridSpec(
        num_scalar_prefetch=1,
        grid=(NB, NI),
        in_specs=[
            pl.BlockSpec((BT, D), x_map),
            pl.BlockSpec((1, BI, D), w_map),
            pl.BlockSpec((1, BI, D), w_map),
            pl.BlockSpec((1, BI, D), w_map),
        ],
        out_specs=pl.BlockSpec((BT, D), o_map),
    )
    out_pad = pl.pallas_call(
        _ffn_kernel,
        grid_spec=grid_spec,
        out_shape=jax.ShapeDtypeStruct((NB * BT, D), jnp.float32),
        compiler_params=pltpu.CompilerParams(
            dimension_semantics=("parallel", "arbitrary"),
            vmem_limit_bytes=100 * 1024 * 1024,
        ),
    )(sched, x_pad, w1, w3, w2)

    out = out_pad[dest].reshape(T, A, D)
    return out


# speedup vs baseline: 1.0254x; 1.0254x over previous
import jax
import jax.numpy as jnp
from jax.experimental import pallas as pl
from jax.experimental.pallas import tpu as pltpu

BT = 512
BI = 704


def _ffn_kernel(sched_ref, oh_ref, x_ref, w1_ref, w3_ref, w2_ref, o_ref,
                xb_s, acc_s):
    b = pl.program_id(0)
    io = pl.program_id(1)
    nb = pl.num_programs(0)
    ni = pl.num_programs(1)
    active = sched_ref[nb + b]

    @pl.when(active == 1)
    def _():
        @pl.when(io == 0)
        def _():
            xb_s[...] = jnp.dot(
                oh_ref[...], x_ref[...],
                preferred_element_type=jnp.float32).astype(jnp.bfloat16)

        xb = xb_s[...]
        w1b = w1_ref[0].astype(jnp.bfloat16)
        w3b = w3_ref[0].astype(jnp.bfloat16)
        w2b = w2_ref[0].astype(jnp.bfloat16)
        dn = (((1,), (1,)), ((), ()))
        h1 = jax.lax.dot_general(xb, w1b, dn, preferred_element_type=jnp.float32)
        h3 = jax.lax.dot_general(xb, w3b, dn, preferred_element_type=jnp.float32)
        h = (h1 * jax.nn.sigmoid(h1) * h3).astype(jnp.bfloat16)
        contrib = jnp.dot(h, w2b, preferred_element_type=jnp.float32)

        @pl.when(io == 0)
        def _():
            acc_s[...] = contrib

        @pl.when(io > 0)
        def _():
            acc_s[...] += contrib

        @pl.when(io == ni - 1)
        def _():
            o_ref[...] = acc_s[...].astype(jnp.bfloat16)


@jax.jit
def kernel(x, expert_indices, w1, w2, w3):
    T, D = x.shape
    A = expert_indices.shape[1]
    E, I, _ = w1.shape
    S = T * A
    NB = S // BT + E
    NI = I // BI

    e_flat = expert_indices.reshape(-1).astype(jnp.int32)
    onehot = (e_flat[:, None] == jnp.arange(E, dtype=jnp.int32)[None, :]
              ).astype(jnp.int32)
    cum = jnp.cumsum(onehot, axis=0)
    counts = cum[-1]
    blocks_per = (counts + BT - 1) // BT
    blocks_cum = jnp.cumsum(blocks_per)
    block_start = blocks_cum - blocks_per
    total_blocks = blocks_cum[-1]

    rank = jnp.take_along_axis(cum, e_flat[:, None], axis=1)[:, 0] - 1
    dest = block_start[e_flat] * BT + rank

    b_ids = jnp.arange(NB, dtype=jnp.int32)
    be_log = jnp.minimum(
        jnp.searchsorted(blocks_cum, b_ids, side="right").astype(jnp.int32), E - 1)

    j = jnp.arange(S, dtype=jnp.int32)
    tok_pad = jnp.full((NB * BT,), T, jnp.int32).at[dest].set(j // A)
    onehot_rows = (tok_pad[:, None] == jnp.arange(T, dtype=jnp.int32)[None, :]
                   ).astype(jnp.bfloat16)
    x_bf = x.astype(jnp.bfloat16)

    half = NB // 2
    nb0 = jnp.minimum((total_blocks + 1) // 2, half)
    nb1 = total_blocks - nb0
    within = jnp.where(b_ids < half, b_ids, b_ids - half)
    base = jnp.where(b_ids < half, 0, nb0)
    half_n = jnp.where(b_ids < half, nb0, nb1)
    act = (within < half_n).astype(jnp.int32)
    last_log = jnp.minimum(base + jnp.maximum(half_n - 1, 0), total_blocks - 1)
    logical = jnp.where(act == 1, base + within, last_log)
    oblk = jnp.where(act == 1, logical, NB - 1)
    sched = jnp.concatenate([be_log[logical], act, logical, oblk])

    def w_map(b, io, s):
        return (s[b], jnp.where(s[NB + b] == 1, io, NI - 1), 0)

    def x_map(b, io, s):
        return (s[2 * NB + b], 0)

    def o_map(b, io, s):
        return (s[3 * NB + b], 0)

    grid_spec = pltpu.PrefetchScalarGridSpec(
        num_scalar_prefetch=1,
        grid=(NB, NI),
        in_specs=[
            pl.BlockSpec((BT, T), x_map),
            pl.BlockSpec((T, D), lambda b, io, s: (0, 0)),
            pl.BlockSpec((1, BI, D), w_map),
            pl.BlockSpec((1, BI, D), w_map),
            pl.BlockSpec((1, BI, D), w_map),
        ],
        out_specs=pl.BlockSpec((BT, D), o_map),
        scratch_shapes=[
            pltpu.VMEM((BT, D), jnp.bfloat16),
            pltpu.VMEM((BT, D), jnp.float32),
        ],
    )
    out_pad = pl.pallas_call(
        _ffn_kernel,
        grid_spec=grid_spec,
        out_shape=jax.ShapeDtypeStruct((NB * BT, D), jnp.bfloat16),
        compiler_params=pltpu.CompilerParams(
            dimension_semantics=("parallel", "arbitrary"),
            vmem_limit_bytes=100 * 1024 * 1024,
        ),
    )(sched, onehot_rows, x_bf, w1, w3, w2)

    out = out_pad[dest].astype(jnp.float32).reshape(T, A, D)
    return out


# speedup vs baseline: 1.0369x; 1.0112x over previous
import jax
import jax.numpy as jnp
from jax.experimental import pallas as pl
from jax.experimental.pallas import tpu as pltpu

BT = 512
BI = 512
BIH = BI // 2


def _ffn_kernel(sched_ref, oh_ref, x_ref, w1lo_ref, w3lo_ref, w2lo_ref,
                w1hi_ref, w3hi_ref, w2hi_ref, o_ref, xb_s, acc_s):
    b = pl.program_id(0)
    io = pl.program_id(1)
    nb = pl.num_programs(0)
    ni = pl.num_programs(1)
    active = sched_ref[nb + b]

    @pl.when(active == 1)
    def _():
        @pl.when(io == 0)
        def _():
            xb_s[...] = jnp.dot(
                oh_ref[...], x_ref[...],
                preferred_element_type=jnp.float32).astype(jnp.bfloat16)

        xb = xb_s[...]
        dn = (((1,), (1,)), ((), ()))
        contrib = None
        for w1_r, w3_r, w2_r in ((w1lo_ref, w3lo_ref, w2lo_ref),
                                 (w1hi_ref, w3hi_ref, w2hi_ref)):
            w1b = w1_r[0].astype(jnp.bfloat16)
            w3b = w3_r[0].astype(jnp.bfloat16)
            w2b = w2_r[0].astype(jnp.bfloat16)
            h1 = jax.lax.dot_general(xb, w1b, dn,
                                     preferred_element_type=jnp.float32)
            h3 = jax.lax.dot_general(xb, w3b, dn,
                                     preferred_element_type=jnp.float32)
            h = (h1 * jax.nn.sigmoid(h1) * h3).astype(jnp.bfloat16)
            c = jnp.dot(h, w2b, preferred_element_type=jnp.float32)
            contrib = c if contrib is None else contrib + c

        @pl.when(io == 0)
        def _():
            acc_s[...] = contrib

        @pl.when(io > 0)
        def _():
            acc_s[...] += contrib

        @pl.when(io == ni - 1)
        def _():
            o_ref[...] = acc_s[...].astype(jnp.bfloat16)


@jax.jit
def kernel(x, expert_indices, w1, w2, w3):
    T, D = x.shape
    A = expert_indices.shape[1]
    E, I, _ = w1.shape
    S = T * A
    NB = S // BT + E
    NI = I // BI

    e_flat = expert_indices.reshape(-1).astype(jnp.int32)
    onehot = (e_flat[:, None] == jnp.arange(E, dtype=jnp.int32)[None, :]
              ).astype(jnp.int32)
    cum = jnp.cumsum(onehot, axis=0)
    counts = cum[-1]
    blocks_per = (counts + BT - 1) // BT
    blocks_cum = jnp.cumsum(blocks_per)
    block_start = blocks_cum - blocks_per
    total_blocks = blocks_cum[-1]

    rank = jnp.take_along_axis(cum, e_flat[:, None], axis=1)[:, 0] - 1
    dest = block_start[e_flat] * BT + rank

    b_ids = jnp.arange(NB, dtype=jnp.int32)
    be_log = jnp.minimum(
        jnp.searchsorted(blocks_cum, b_ids, side="right").astype(jnp.int32), E - 1)

    j = jnp.arange(S, dtype=jnp.int32)
    tok_pad = jnp.full((NB * BT,), T, jnp.int32).at[dest].set(j // A)
    onehot_rows = (tok_pad[:, None] == jnp.arange(T, dtype=jnp.int32)[None, :]
                   ).astype(jnp.bfloat16)
    x_bf = x.astype(jnp.bfloat16)

    half = NB // 2
    nb0 = jnp.minimum((total_blocks + 1) // 2, half)
    nb1 = total_blocks - nb0
    within = jnp.where(b_ids < half, b_ids, b_ids - half)
    base = jnp.where(b_ids < half, 0, nb0)
    half_n = jnp.where(b_ids < half, nb0, nb1)
    act = (within < half_n).astype(jnp.int32)
    last_log = jnp.minimum(base + jnp.maximum(half_n - 1, 0), total_blocks - 1)
    logical = jnp.where(act == 1, base + within, last_log)
    oblk = jnp.where(act == 1, logical, NB - 1)
    sched = jnp.concatenate([be_log[logical], act, logical, oblk])

    def w_map_lo(b, io, s):
        return (s[b], jnp.where(s[NB + b] == 1, 2 * io, 2 * NI - 2), 0)

    def w_map_hi(b, io, s):
        return (s[b], jnp.where(s[NB + b] == 1, 2 * io + 1, 2 * NI - 1), 0)

    def x_map(b, io, s):
        return (s[2 * NB + b], 0)

    def o_map(b, io, s):
        return (s[3 * NB + b], 0)

    grid_spec = pltpu.PrefetchScalarGridSpec(
        num_scalar_prefetch=1,
        grid=(NB, NI),
        in_specs=[
            pl.BlockSpec((BT, T), x_map),
            pl.BlockSpec((T, D), lambda b, io, s: (0, 0)),
            pl.BlockSpec((1, BIH, D), w_map_lo),
            pl.BlockSpec((1, BIH, D), w_map_lo),
            pl.BlockSpec((1, BIH, D), w_map_lo),
            pl.BlockSpec((1, BIH, D), w_map_hi),
            pl.BlockSpec((1, BIH, D), w_map_hi),
            pl.BlockSpec((1, BIH, D), w_map_hi),
        ],
        out_specs=pl.BlockSpec((BT, D), o_map),
        scratch_shapes=[
            pltpu.VMEM((BT, D), jnp.bfloat16),
            pltpu.VMEM((BT, D), jnp.float32),
        ],
    )
    out_pad = pl.pallas_call(
        _ffn_kernel,
        grid_spec=grid_spec,
        out_shape=jax.ShapeDtypeStruct((NB * BT, D), jnp.bfloat16),
        compiler_params=pltpu.CompilerParams(
            dimension_semantics=("parallel", "arbitrary"),
            vmem_limit_bytes=100 * 1024 * 1024,
        ),
    )(sched, onehot_rows, x_bf, w1, w3, w2, w1, w3, w2)

    out = out_pad[dest].astype(jnp.float32).reshape(T, A, D)
    return out
